# baseline (speedup 1.0000x reference)
import functools

import jax
import jax.numpy as jnp
from jax.experimental import pallas as pl
from jax.experimental.pallas import tpu as pltpu


def _fused_kernel(x_ref, a_ref, bg_ref, wg_ref, wd_ref, bd_ref, out_ref,
                  hall_ref, *, PA, B, H):
    s = pl.program_id(0)

    @pl.when(s < PA)
    def _phase_a():
        wg = wg_ref[...].astype(jnp.bfloat16)
        h0 = jax.lax.dot_general(
            x_ref[0].astype(jnp.bfloat16), wg,
            dimension_numbers=(((0,), (0,)), ((), ())),
            preferred_element_type=jnp.float32,
        )
        h1 = jax.lax.dot_general(
            x_ref[1].astype(jnp.bfloat16), wg,
            dimension_numbers=(((0,), (0,)), ((), ())),
            preferred_element_type=jnp.float32,
        )
        hall_ref[s] = jnp.concatenate([h0, h1], axis=1).astype(jnp.bfloat16)

    @pl.when(s >= PA)
    def _phase_b():
        a_blk = a_ref[...].astype(jnp.bfloat16)
        wd = wd_ref[...].astype(jnp.bfloat16)
        for j in range(B // 2):
            g = jnp.dot(a_blk, hall_ref[j],
                        preferred_element_type=jnp.float32)
            for k in range(2):
                gb = jnp.maximum(g[:, k * H:(k + 1) * H] + bg_ref[...], 0.0)
                ob = jnp.dot(gb.astype(jnp.bfloat16), wd,
                             preferred_element_type=jnp.float32)
                out_ref[2 * j + k] = jnp.maximum(ob + bd_ref[...], 0.0)


def kernel(x, a, W_gcn, b_gcn, W_d, b_d):
    B, T, N = x.shape
    H = W_gcn.shape[1]
    bg = b_gcn.reshape(1, H)
    bd = b_d.reshape(1, T)

    PA = B // 2
    BA = 256
    PB = N // BA

    return pl.pallas_call(
        functools.partial(_fused_kernel, PA=PA, B=B, H=H),
        grid=(PA + PB,),
        in_specs=[
            pl.BlockSpec((2, T, N), lambda s: (jnp.minimum(s, PA - 1), 0, 0)),
            pl.BlockSpec((BA, N), lambda s: (jnp.maximum(s - PA, 0), 0)),
            pl.BlockSpec((1, H), lambda s: (0, 0)),
            pl.BlockSpec((T, H), lambda s: (0, 0)),
            pl.BlockSpec((H, T), lambda s: (0, 0)),
            pl.BlockSpec((1, T), lambda s: (0, 0)),
        ],
        out_specs=pl.BlockSpec(
            (B, BA, T), lambda s: (0, jnp.maximum(s - PA, 0), 0)),
        out_shape=jax.ShapeDtypeStruct((B, N, T), jnp.float32),
        scratch_shapes=[pltpu.VMEM((PA, N, 2 * H), jnp.bfloat16)],
    )(x, a, bg, W_gcn, W_d, bd)
